# baseline (speedup 1.0000x reference)
import functools

import jax
import jax.numpy as jnp
from jax import lax
from jax.experimental import pallas as pl
from jax.experimental.pallas import tpu as pltpu
from jax.experimental.pallas import tpu_sc as plsc

N_NODES = 10000
N_EDGES = 320000
D = 128
C_OUT = 16

NPAD = 10240
N_CORES = 2
N_SUB = 16
CHUNK = 128
CHUNKS_PER_TILE = 80
PHASES = 2
PH_CHUNKS = CHUNKS_PER_TILE // PHASES
EPAD = N_CORES * N_SUB * CHUNKS_PER_TILE * CHUNK
ROWS_PER_TILE = NPAD // N_SUB
HOPS = ROWS_PER_TILE // CHUNK
PAD_IDX = N_NODES
W2 = 32

_mesh = plsc.VectorSubcoreMesh(core_axis_name="c", subcore_axis_name="s")


@functools.partial(
    pl.kernel,
    mesh=_mesh,
    out_type=jax.ShapeDtypeStruct((N_CORES, NPAD, D), jnp.float32),
    scratch_types=[
        pltpu.VMEM((CHUNKS_PER_TILE, CHUNK), jnp.int32),
        pltpu.VMEM((CHUNK, D), jnp.float32),
        pltpu.VMEM((CHUNK, D), jnp.float32),
        pltpu.VMEM_SHARED((NPAD, D), jnp.float32),
    ],
)
def _degrees_kernel(i_hbm, zeros_hbm, ones_hbm, out_hbm,
                    idx, ones_v, stage, acc):
    c = lax.axis_index("c")
    s = lax.axis_index("s")
    pltpu.sync_copy(i_hbm.at[c, s], idx)
    pltpu.sync_copy(ones_hbm, ones_v)
    pltpu.sync_copy(zeros_hbm, stage)
    base = s * ROWS_PER_TILE
    for h in range(HOPS):
        pltpu.sync_copy(stage, acc.at[pl.ds(base + h * CHUNK, CHUNK)])
    plsc.subcore_barrier()

    @pl.loop(0, CHUNKS_PER_TILE)
    def _(j):
        pltpu.sync_copy(ones_v, acc.at[idx.at[j]], add=True)

    plsc.subcore_barrier()
    for h in range(HOPS):
        off = base + h * CHUNK
        pltpu.sync_copy(acc.at[pl.ds(off, CHUNK)], stage)
        pltpu.sync_copy(stage, out_hbm.at[c, pl.ds(off, CHUNK)])


TOTAL_CHUNKS = EPAD // CHUNK
KA = 160
KB = 0
KP = 32
A_TOTAL = N_SUB * KA
assert N_SUB * (KA + KB) == TOTAL_CHUNKS


@functools.partial(
    pl.kernel,
    mesh=_mesh,
    out_type=jax.ShapeDtypeStruct((N_CORES, NPAD, D), jnp.float32),
    scratch_types=[
        pltpu.VMEM((KP, CHUNK), jnp.int32),
        pltpu.VMEM((KP, CHUNK), jnp.int32),
        pltpu.VMEM((2, CHUNK, D), jnp.float32),
        pltpu.VMEM_SHARED((NPAD, D), jnp.float32),
        pltpu.SemaphoreType.DMA,
        pltpu.SemaphoreType.DMA,
    ],
)
def _agg128(table_hbm, s_hbm, r_hbm, zeros_hbm, out_hbm,
            sidx, ridx, rows, acc, gsem0, gsem1):
    c = lax.axis_index("c")
    s = lax.axis_index("s")
    base = s * ROWS_PER_TILE
    pltpu.sync_copy(zeros_hbm, rows.at[0])
    for h in range(HOPS):
        pltpu.sync_copy(rows.at[0], acc.at[pl.ds(base + h * CHUNK, CHUNK)])
    plsc.subcore_barrier()

    start = jnp.where(c == 0, s * KA, A_TOTAL + s * KB)
    nphases = jnp.where(c == 0, KA // KP, KB // KP)

    for p in range(KA // KP):
        @pl.when(p < nphases)
        def _():
            off = start + p * KP
            pltpu.sync_copy(s_hbm.at[pl.ds(off, KP)], sidx)
            pltpu.sync_copy(r_hbm.at[pl.ds(off, KP)], ridx)

            pltpu.async_copy(table_hbm.at[sidx.at[0]], rows.at[0], gsem0)

            @pl.loop(0, KP, step=2)
            def _(j):
                pltpu.async_copy(table_hbm.at[sidx.at[j + 1]], rows.at[1],
                                 gsem1)
                pltpu.make_async_copy(table_hbm.at[sidx.at[j]], rows.at[0],
                                      gsem0).wait()
                pltpu.sync_copy(rows.at[0], acc.at[ridx.at[j]], add=True)

                @pl.when(j + 2 < KP)
                def _():
                    pltpu.async_copy(table_hbm.at[sidx.at[j + 2]], rows.at[0],
                                     gsem0)

                pltpu.make_async_copy(table_hbm.at[sidx.at[j + 1]],
                                      rows.at[1], gsem1).wait()
                pltpu.sync_copy(rows.at[1], acc.at[ridx.at[j + 1]], add=True)

    plsc.subcore_barrier()
    for h in range(HOPS):
        off = base + h * CHUNK
        pltpu.sync_copy(acc.at[pl.ds(off, CHUNK)], rows.at[0])
        pltpu.sync_copy(rows.at[0], out_hbm.at[c, pl.ds(off, CHUNK)])


_BLK = 1024


def _tc_encode_body(x_ref, w0_ref, b0_ref, degs_ref, degr_ref, y_ref, sr_ref):
    ds = (degs_ref[0] + degs_ref[1])[:, 0:1]
    dr = (degr_ref[0] + degr_ref[1])[:, 0:1]
    ss = lax.rsqrt(jnp.maximum(ds, 1.0))
    sr = lax.rsqrt(jnp.maximum(dr, 1.0))
    y = jnp.dot(x_ref[...], w0_ref[...], preferred_element_type=jnp.float32)
    y = (y + b0_ref[...]) * ss
    row = pl.program_id(0) * _BLK + lax.broadcasted_iota(jnp.int32, (_BLK, 1), 0)
    y_ref[...] = jnp.where(row < N_NODES, y, 0.0)
    sr_ref[...] = sr


def _tc_encode(x_pad, w0, b0, deg_s, deg_r):
    return pl.pallas_call(
        _tc_encode_body,
        grid=(NPAD // _BLK,),
        in_specs=[
            pl.BlockSpec((_BLK, D), lambda i: (i, 0)),
            pl.BlockSpec((D, D), lambda i: (0, 0)),
            pl.BlockSpec((1, D), lambda i: (0, 0)),
            pl.BlockSpec((N_CORES, _BLK, D), lambda i: (0, i, 0)),
            pl.BlockSpec((N_CORES, _BLK, D), lambda i: (0, i, 0)),
        ],
        out_specs=[
            pl.BlockSpec((_BLK, D), lambda i: (i, 0)),
            pl.BlockSpec((_BLK, 1), lambda i: (i, 0)),
        ],
        out_shape=[
            jax.ShapeDtypeStruct((NPAD, D), jnp.float32),
            jax.ShapeDtypeStruct((NPAD, 1), jnp.float32),
        ],
    )(x_pad, w0, b0, deg_s, deg_r)


def _tc_decode_body(acc_ref, sr_ref, w1_ref, b1_ref, z_ref):
    h = (acc_ref[0] + acc_ref[1]) * sr_ref[...]
    h = jnp.where(h >= 0, h, 0.01 * h)
    z = jnp.dot(h, w1_ref[...], preferred_element_type=jnp.float32) + b1_ref[...]
    z = 1.0 / (1.0 + jnp.exp(-z))
    row = pl.program_id(0) * _BLK + lax.broadcasted_iota(jnp.int32, (_BLK, 1), 0)
    col = lax.broadcasted_iota(jnp.int32, (_BLK, D), 1)
    z_ref[...] = jnp.where((row < N_NODES) & (col < C_OUT), z, 0.0)


def _tc_decode(acc, sr, w1, b1):
    return pl.pallas_call(
        _tc_decode_body,
        grid=(NPAD // _BLK,),
        in_specs=[
            pl.BlockSpec((N_CORES, _BLK, D), lambda i: (0, i, 0)),
            pl.BlockSpec((_BLK, 1), lambda i: (i, 0)),
            pl.BlockSpec((D, D), lambda i: (0, 0)),
            pl.BlockSpec((1, D), lambda i: (0, 0)),
        ],
        out_specs=pl.BlockSpec((_BLK, D), lambda i: (i, 0)),
        out_shape=jax.ShapeDtypeStruct((NPAD, D), jnp.float32),
    )(acc, sr, w1, b1)


def _tc_combine_body(acc_ref, out_ref):
    out_ref[...] = (acc_ref[0] + acc_ref[1])[:, :C_OUT]


def _tc_combine(acc):
    return pl.pallas_call(
        _tc_combine_body,
        grid=(NPAD // _BLK,),
        in_specs=[pl.BlockSpec((N_CORES, _BLK, D), lambda i: (0, i, 0))],
        out_specs=pl.BlockSpec((_BLK, C_OUT), lambda i: (i, 0)),
        out_shape=jax.ShapeDtypeStruct((NPAD, C_OUT), jnp.float32),
    )(acc)


def kernel(x, edge_index, W0, b0, W1, b1):
    senders = edge_index[0].astype(jnp.int32)
    receivers = edge_index[1].astype(jnp.int32)
    pad = EPAD - N_EDGES
    senders = jnp.pad(senders, (0, pad), constant_values=PAD_IDX)
    receivers = jnp.pad(receivers, (0, pad), constant_values=PAD_IDX)
    s_slab = senders.reshape(N_CORES, N_SUB, CHUNKS_PER_TILE, CHUNK)
    r_slab = receivers.reshape(N_CORES, N_SUB, CHUNKS_PER_TILE, CHUNK)
    s_flat = senders.reshape(TOTAL_CHUNKS, CHUNK)
    r_flat = receivers.reshape(TOTAL_CHUNKS, CHUNK)

    x_pad = jnp.pad(x, ((0, NPAD - N_NODES), (0, 0)))
    ones_row = jnp.ones((CHUNK, D), jnp.float32)
    zeros_row = jnp.zeros((CHUNK, D), jnp.float32)
    w1_pad = jnp.pad(W1, ((0, 0), (0, D - C_OUT)))
    b1_pad = jnp.pad(b1.reshape(1, C_OUT), ((0, 0), (0, D - C_OUT)))

    deg_s = _degrees_kernel(s_slab, zeros_row, ones_row)
    deg_r = _degrees_kernel(r_slab, zeros_row, ones_row)
    y, sr = _tc_encode(x_pad, W0, b0.reshape(1, D), deg_s, deg_r)
    acc1 = _agg128(y, s_flat, r_flat, zeros_row)
    z = _tc_decode(acc1, sr, w1_pad, b1_pad)
    acc2 = _agg128(z, s_flat, r_flat, zeros_row)
    out = _tc_combine(acc2)
    return out[:N_NODES]


# speedup vs baseline: 1.3301x; 1.3301x over previous
import functools

import jax
import jax.numpy as jnp
from jax import lax
from jax.experimental import pallas as pl
from jax.experimental.pallas import tpu as pltpu
from jax.experimental.pallas import tpu_sc as plsc

N_NODES = 10000
N_EDGES = 320000
D = 128
C_OUT = 16

NPAD = 10240
N_CORES = 2
N_SUB = 16
CHUNK = 128
CHUNKS_PER_TILE = 80
PHASES = 2
PH_CHUNKS = CHUNKS_PER_TILE // PHASES
EPAD = N_CORES * N_SUB * CHUNKS_PER_TILE * CHUNK
ROWS_PER_TILE = NPAD // N_SUB
HOPS = ROWS_PER_TILE // CHUNK
PAD_IDX = N_NODES
W2 = 32

_mesh = plsc.VectorSubcoreMesh(core_axis_name="c", subcore_axis_name="s")


@functools.partial(
    pl.kernel,
    mesh=_mesh,
    out_type=jax.ShapeDtypeStruct((N_CORES, NPAD, D), jnp.float32),
    scratch_types=[
        pltpu.VMEM((CHUNKS_PER_TILE, CHUNK), jnp.int32),
        pltpu.VMEM((CHUNK, D), jnp.float32),
        pltpu.VMEM((CHUNK, D), jnp.float32),
        pltpu.VMEM_SHARED((NPAD, D), jnp.float32),
    ],
)
def _degrees_kernel(i_hbm, zeros_hbm, ones_hbm, out_hbm,
                    idx, ones_v, stage, acc):
    c = lax.axis_index("c")
    s = lax.axis_index("s")
    pltpu.sync_copy(i_hbm.at[c, s], idx)
    pltpu.sync_copy(ones_hbm, ones_v)
    pltpu.sync_copy(zeros_hbm, stage)
    base = s * ROWS_PER_TILE
    for h in range(HOPS):
        pltpu.sync_copy(stage, acc.at[pl.ds(base + h * CHUNK, CHUNK)])
    plsc.subcore_barrier()

    @pl.loop(0, CHUNKS_PER_TILE)
    def _(j):
        pltpu.sync_copy(ones_v, acc.at[idx.at[j]], add=True)

    plsc.subcore_barrier()
    for h in range(HOPS):
        off = base + h * CHUNK
        pltpu.sync_copy(acc.at[pl.ds(off, CHUNK)], stage)
        pltpu.sync_copy(stage, out_hbm.at[c, pl.ds(off, CHUNK)])


TOTAL_CHUNKS = EPAD // CHUNK
KA = 144
KB = 16
KP = 16
A_TOTAL = N_SUB * KA
assert N_SUB * (KA + KB) == TOTAL_CHUNKS


@functools.partial(
    pl.kernel,
    mesh=_mesh,
    out_type=jax.ShapeDtypeStruct((N_CORES, NPAD, D), jnp.float32),
    scratch_types=[
        pltpu.VMEM((KP, CHUNK), jnp.int32),
        pltpu.VMEM((KP, CHUNK), jnp.int32),
        pltpu.VMEM((2, CHUNK, D), jnp.float32),
        pltpu.VMEM_SHARED((NPAD, D), jnp.float32),
        pltpu.SemaphoreType.DMA,
        pltpu.SemaphoreType.DMA,
    ],
)
def _agg128(table_hbm, s_hbm, r_hbm, zeros_hbm, out_hbm,
            sidx, ridx, rows, acc, gsem0, gsem1):
    c = lax.axis_index("c")
    s = lax.axis_index("s")
    base = s * ROWS_PER_TILE
    pltpu.sync_copy(zeros_hbm, rows.at[0])
    for h in range(HOPS):
        pltpu.sync_copy(rows.at[0], acc.at[pl.ds(base + h * CHUNK, CHUNK)])
    plsc.subcore_barrier()

    start = jnp.where(c == 0, s * KA, A_TOTAL + s * KB)
    nphases = jnp.where(c == 0, KA // KP, KB // KP)

    for p in range(KA // KP):
        @pl.when(p < nphases)
        def _():
            off = start + p * KP
            pltpu.sync_copy(s_hbm.at[pl.ds(off, KP)], sidx)
            pltpu.sync_copy(r_hbm.at[pl.ds(off, KP)], ridx)

            pltpu.async_copy(table_hbm.at[sidx.at[0]], rows.at[0], gsem0)

            @pl.loop(0, KP, step=2)
            def _(j):
                pltpu.async_copy(table_hbm.at[sidx.at[j + 1]], rows.at[1],
                                 gsem1)
                pltpu.make_async_copy(table_hbm.at[sidx.at[j]], rows.at[0],
                                      gsem0).wait()
                pltpu.sync_copy(rows.at[0], acc.at[ridx.at[j]], add=True)

                @pl.when(j + 2 < KP)
                def _():
                    pltpu.async_copy(table_hbm.at[sidx.at[j + 2]], rows.at[0],
                                     gsem0)

                pltpu.make_async_copy(table_hbm.at[sidx.at[j + 1]],
                                      rows.at[1], gsem1).wait()
                pltpu.sync_copy(rows.at[1], acc.at[ridx.at[j + 1]], add=True)

    plsc.subcore_barrier()
    for h in range(HOPS):
        off = base + h * CHUNK
        pltpu.sync_copy(acc.at[pl.ds(off, CHUNK)], rows.at[0])
        pltpu.sync_copy(rows.at[0], out_hbm.at[c, pl.ds(off, CHUNK)])


_BLK = 1024


def _tc_encode_body(x_ref, w0_ref, b0_ref, degs_ref, degr_ref, y_ref, sr_ref):
    ds = (degs_ref[0] + degs_ref[1])[:, 0:1]
    dr = (degr_ref[0] + degr_ref[1])[:, 0:1]
    ss = lax.rsqrt(jnp.maximum(ds, 1.0))
    sr = lax.rsqrt(jnp.maximum(dr, 1.0))
    y = jnp.dot(x_ref[...], w0_ref[...], preferred_element_type=jnp.float32)
    y = (y + b0_ref[...]) * ss
    row = pl.program_id(0) * _BLK + lax.broadcasted_iota(jnp.int32, (_BLK, 1), 0)
    y_ref[...] = jnp.where(row < N_NODES, y, 0.0)
    sr_ref[...] = sr


def _tc_encode(x_pad, w0, b0, deg_s, deg_r):
    return pl.pallas_call(
        _tc_encode_body,
        grid=(NPAD // _BLK,),
        in_specs=[
            pl.BlockSpec((_BLK, D), lambda i: (i, 0)),
            pl.BlockSpec((D, D), lambda i: (0, 0)),
            pl.BlockSpec((1, D), lambda i: (0, 0)),
            pl.BlockSpec((N_CORES, _BLK, D), lambda i: (0, i, 0)),
            pl.BlockSpec((N_CORES, _BLK, D), lambda i: (0, i, 0)),
        ],
        out_specs=[
            pl.BlockSpec((_BLK, D), lambda i: (i, 0)),
            pl.BlockSpec((_BLK, 1), lambda i: (i, 0)),
        ],
        out_shape=[
            jax.ShapeDtypeStruct((NPAD, D), jnp.float32),
            jax.ShapeDtypeStruct((NPAD, 1), jnp.float32),
        ],
    )(x_pad, w0, b0, deg_s, deg_r)


def _tc_decode_body(acc_ref, sr_ref, w1_ref, b1_ref, z_ref):
    h = (acc_ref[0] + acc_ref[1]) * sr_ref[...]
    h = jnp.where(h >= 0, h, 0.01 * h)
    z = jnp.dot(h, w1_ref[...], preferred_element_type=jnp.float32) + b1_ref[...]
    z = 1.0 / (1.0 + jnp.exp(-z))
    row = pl.program_id(0) * _BLK + lax.broadcasted_iota(jnp.int32, (_BLK, 1), 0)
    col = lax.broadcasted_iota(jnp.int32, (_BLK, D), 1)
    z_ref[...] = jnp.where((row < N_NODES) & (col < C_OUT), z, 0.0)


def _tc_decode(acc, sr, w1, b1):
    return pl.pallas_call(
        _tc_decode_body,
        grid=(NPAD // _BLK,),
        in_specs=[
            pl.BlockSpec((N_CORES, _BLK, D), lambda i: (0, i, 0)),
            pl.BlockSpec((_BLK, 1), lambda i: (i, 0)),
            pl.BlockSpec((D, D), lambda i: (0, 0)),
            pl.BlockSpec((1, D), lambda i: (0, 0)),
        ],
        out_specs=pl.BlockSpec((_BLK, D), lambda i: (i, 0)),
        out_shape=jax.ShapeDtypeStruct((NPAD, D), jnp.float32),
    )(acc, sr, w1, b1)


def _tc_combine_body(acc_ref, out_ref):
    out_ref[...] = (acc_ref[0] + acc_ref[1])[:, :C_OUT]


def _tc_combine(acc):
    return pl.pallas_call(
        _tc_combine_body,
        grid=(NPAD // _BLK,),
        in_specs=[pl.BlockSpec((N_CORES, _BLK, D), lambda i: (0, i, 0))],
        out_specs=pl.BlockSpec((_BLK, C_OUT), lambda i: (i, 0)),
        out_shape=jax.ShapeDtypeStruct((NPAD, C_OUT), jnp.float32),
    )(acc)


def kernel(x, edge_index, W0, b0, W1, b1):
    senders = edge_index[0].astype(jnp.int32)
    receivers = edge_index[1].astype(jnp.int32)
    pad = EPAD - N_EDGES
    senders = jnp.pad(senders, (0, pad), constant_values=PAD_IDX)
    receivers = jnp.pad(receivers, (0, pad), constant_values=PAD_IDX)
    s_slab = senders.reshape(N_CORES, N_SUB, CHUNKS_PER_TILE, CHUNK)
    r_slab = receivers.reshape(N_CORES, N_SUB, CHUNKS_PER_TILE, CHUNK)
    s_flat = senders.reshape(TOTAL_CHUNKS, CHUNK)
    r_flat = receivers.reshape(TOTAL_CHUNKS, CHUNK)

    x_pad = jnp.pad(x, ((0, NPAD - N_NODES), (0, 0)))
    ones_row = jnp.ones((CHUNK, D), jnp.float32)
    zeros_row = jnp.zeros((CHUNK, D), jnp.float32)
    w1_pad = jnp.pad(W1, ((0, 0), (0, D - C_OUT)))
    b1_pad = jnp.pad(b1.reshape(1, C_OUT), ((0, 0), (0, D - C_OUT)))

    deg_s = _degrees_kernel(s_slab, zeros_row, ones_row)
    deg_r = _degrees_kernel(r_slab, zeros_row, ones_row)
    y, sr = _tc_encode(x_pad, W0, b0.reshape(1, D), deg_s, deg_r)
    acc1 = _agg128(y, s_flat, r_flat, zeros_row)
    z = _tc_decode(acc1, sr, w1_pad, b1_pad)
    acc2 = _agg128(z, s_flat, r_flat, zeros_row)
    out = _tc_combine(acc2)
    return out[:N_NODES]


# speedup vs baseline: 1.3429x; 1.0096x over previous
import functools

import jax
import jax.numpy as jnp
from jax import lax
from jax.experimental import pallas as pl
from jax.experimental.pallas import tpu as pltpu
from jax.experimental.pallas import tpu_sc as plsc

N_NODES = 10000
N_EDGES = 320000
D = 128
C_OUT = 16

NPAD = 10240
N_CORES = 2
N_SUB = 16
CHUNK = 128
CHUNKS_PER_TILE = 80
PHASES = 2
PH_CHUNKS = CHUNKS_PER_TILE // PHASES
EPAD = N_CORES * N_SUB * CHUNKS_PER_TILE * CHUNK
ROWS_PER_TILE = NPAD // N_SUB
HOPS = ROWS_PER_TILE // CHUNK
PAD_IDX = N_NODES
W2 = 32

_mesh = plsc.VectorSubcoreMesh(core_axis_name="c", subcore_axis_name="s")


@functools.partial(
    pl.kernel,
    mesh=_mesh,
    out_type=jax.ShapeDtypeStruct((N_CORES, NPAD, D), jnp.float32),
    scratch_types=[
        pltpu.VMEM((CHUNKS_PER_TILE, CHUNK), jnp.int32),
        pltpu.VMEM((CHUNK, D), jnp.float32),
        pltpu.VMEM((CHUNK, D), jnp.float32),
        pltpu.VMEM_SHARED((NPAD, D), jnp.float32),
    ],
)
def _degrees_kernel(i_hbm, zeros_hbm, ones_hbm, out_hbm,
                    idx, ones_v, stage, acc):
    c = lax.axis_index("c")
    s = lax.axis_index("s")
    pltpu.sync_copy(i_hbm.at[c, s], idx)
    pltpu.sync_copy(ones_hbm, ones_v)
    pltpu.sync_copy(zeros_hbm, stage)
    base = s * ROWS_PER_TILE
    for h in range(HOPS):
        pltpu.sync_copy(stage, acc.at[pl.ds(base + h * CHUNK, CHUNK)])
    plsc.subcore_barrier()

    @pl.loop(0, CHUNKS_PER_TILE)
    def _(j):
        pltpu.sync_copy(ones_v, acc.at[idx.at[j]], add=True)

    plsc.subcore_barrier()
    for h in range(HOPS):
        off = base + h * CHUNK
        pltpu.sync_copy(acc.at[pl.ds(off, CHUNK)], stage)
        pltpu.sync_copy(stage, out_hbm.at[c, pl.ds(off, CHUNK)])


TOTAL_CHUNKS = EPAD // CHUNK
KA = 152
KB = 8
KP = 8
A_TOTAL = N_SUB * KA
assert N_SUB * (KA + KB) == TOTAL_CHUNKS


@functools.partial(
    pl.kernel,
    mesh=_mesh,
    out_type=jax.ShapeDtypeStruct((N_CORES, NPAD, D), jnp.float32),
    scratch_types=[
        pltpu.VMEM((KP, CHUNK), jnp.int32),
        pltpu.VMEM((KP, CHUNK), jnp.int32),
        pltpu.VMEM((2, CHUNK, D), jnp.float32),
        pltpu.VMEM_SHARED((NPAD, D), jnp.float32),
        pltpu.SemaphoreType.DMA,
        pltpu.SemaphoreType.DMA,
    ],
)
def _agg128(table_hbm, s_hbm, r_hbm, zeros_hbm, out_hbm,
            sidx, ridx, rows, acc, gsem0, gsem1):
    c = lax.axis_index("c")
    s = lax.axis_index("s")
    base = s * ROWS_PER_TILE
    pltpu.sync_copy(zeros_hbm, rows.at[0])
    for h in range(HOPS):
        pltpu.sync_copy(rows.at[0], acc.at[pl.ds(base + h * CHUNK, CHUNK)])
    plsc.subcore_barrier()

    start = jnp.where(c == 0, s * KA, A_TOTAL + s * KB)
    nphases = jnp.where(c == 0, KA // KP, KB // KP)

    for p in range(KA // KP):
        @pl.when(p < nphases)
        def _():
            off = start + p * KP
            pltpu.sync_copy(s_hbm.at[pl.ds(off, KP)], sidx)
            pltpu.sync_copy(r_hbm.at[pl.ds(off, KP)], ridx)

            pltpu.async_copy(table_hbm.at[sidx.at[0]], rows.at[0], gsem0)

            @pl.loop(0, KP, step=2)
            def _(j):
                pltpu.async_copy(table_hbm.at[sidx.at[j + 1]], rows.at[1],
                                 gsem1)
                pltpu.make_async_copy(table_hbm.at[sidx.at[j]], rows.at[0],
                                      gsem0).wait()
                pltpu.sync_copy(rows.at[0], acc.at[ridx.at[j]], add=True)

                @pl.when(j + 2 < KP)
                def _():
                    pltpu.async_copy(table_hbm.at[sidx.at[j + 2]], rows.at[0],
                                     gsem0)

                pltpu.make_async_copy(table_hbm.at[sidx.at[j + 1]],
                                      rows.at[1], gsem1).wait()
                pltpu.sync_copy(rows.at[1], acc.at[ridx.at[j + 1]], add=True)

    plsc.subcore_barrier()
    for h in range(HOPS):
        off = base + h * CHUNK
        pltpu.sync_copy(acc.at[pl.ds(off, CHUNK)], rows.at[0])
        pltpu.sync_copy(rows.at[0], out_hbm.at[c, pl.ds(off, CHUNK)])


_BLK = 1024


def _tc_encode_body(x_ref, w0_ref, b0_ref, degs_ref, degr_ref, y_ref, sr_ref):
    ds = (degs_ref[0] + degs_ref[1])[:, 0:1]
    dr = (degr_ref[0] + degr_ref[1])[:, 0:1]
    ss = lax.rsqrt(jnp.maximum(ds, 1.0))
    sr = lax.rsqrt(jnp.maximum(dr, 1.0))
    y = jnp.dot(x_ref[...], w0_ref[...], preferred_element_type=jnp.float32)
    y = (y + b0_ref[...]) * ss
    row = pl.program_id(0) * _BLK + lax.broadcasted_iota(jnp.int32, (_BLK, 1), 0)
    y_ref[...] = jnp.where(row < N_NODES, y, 0.0)
    sr_ref[...] = sr


def _tc_encode(x_pad, w0, b0, deg_s, deg_r):
    return pl.pallas_call(
        _tc_encode_body,
        grid=(NPAD // _BLK,),
        in_specs=[
            pl.BlockSpec((_BLK, D), lambda i: (i, 0)),
            pl.BlockSpec((D, D), lambda i: (0, 0)),
            pl.BlockSpec((1, D), lambda i: (0, 0)),
            pl.BlockSpec((N_CORES, _BLK, D), lambda i: (0, i, 0)),
            pl.BlockSpec((N_CORES, _BLK, D), lambda i: (0, i, 0)),
        ],
        out_specs=[
            pl.BlockSpec((_BLK, D), lambda i: (i, 0)),
            pl.BlockSpec((_BLK, 1), lambda i: (i, 0)),
        ],
        out_shape=[
            jax.ShapeDtypeStruct((NPAD, D), jnp.float32),
            jax.ShapeDtypeStruct((NPAD, 1), jnp.float32),
        ],
    )(x_pad, w0, b0, deg_s, deg_r)


def _tc_decode_body(acc_ref, sr_ref, w1_ref, b1_ref, z_ref):
    h = (acc_ref[0] + acc_ref[1]) * sr_ref[...]
    h = jnp.where(h >= 0, h, 0.01 * h)
    z = jnp.dot(h, w1_ref[...], preferred_element_type=jnp.float32) + b1_ref[...]
    z = 1.0 / (1.0 + jnp.exp(-z))
    row = pl.program_id(0) * _BLK + lax.broadcasted_iota(jnp.int32, (_BLK, 1), 0)
    col = lax.broadcasted_iota(jnp.int32, (_BLK, D), 1)
    z_ref[...] = jnp.where((row < N_NODES) & (col < C_OUT), z, 0.0)


def _tc_decode(acc, sr, w1, b1):
    return pl.pallas_call(
        _tc_decode_body,
        grid=(NPAD // _BLK,),
        in_specs=[
            pl.BlockSpec((N_CORES, _BLK, D), lambda i: (0, i, 0)),
            pl.BlockSpec((_BLK, 1), lambda i: (i, 0)),
            pl.BlockSpec((D, D), lambda i: (0, 0)),
            pl.BlockSpec((1, D), lambda i: (0, 0)),
        ],
        out_specs=pl.BlockSpec((_BLK, D), lambda i: (i, 0)),
        out_shape=jax.ShapeDtypeStruct((NPAD, D), jnp.float32),
    )(acc, sr, w1, b1)


def _tc_combine_body(acc_ref, out_ref):
    out_ref[...] = (acc_ref[0] + acc_ref[1])[:, :C_OUT]


def _tc_combine(acc):
    return pl.pallas_call(
        _tc_combine_body,
        grid=(NPAD // _BLK,),
        in_specs=[pl.BlockSpec((N_CORES, _BLK, D), lambda i: (0, i, 0))],
        out_specs=pl.BlockSpec((_BLK, C_OUT), lambda i: (i, 0)),
        out_shape=jax.ShapeDtypeStruct((NPAD, C_OUT), jnp.float32),
    )(acc)


def kernel(x, edge_index, W0, b0, W1, b1):
    senders = edge_index[0].astype(jnp.int32)
    receivers = edge_index[1].astype(jnp.int32)
    pad = EPAD - N_EDGES
    senders = jnp.pad(senders, (0, pad), constant_values=PAD_IDX)
    receivers = jnp.pad(receivers, (0, pad), constant_values=PAD_IDX)
    s_slab = senders.reshape(N_CORES, N_SUB, CHUNKS_PER_TILE, CHUNK)
    r_slab = receivers.reshape(N_CORES, N_SUB, CHUNKS_PER_TILE, CHUNK)
    s_flat = senders.reshape(TOTAL_CHUNKS, CHUNK)
    r_flat = receivers.reshape(TOTAL_CHUNKS, CHUNK)

    x_pad = jnp.pad(x, ((0, NPAD - N_NODES), (0, 0)))
    ones_row = jnp.ones((CHUNK, D), jnp.float32)
    zeros_row = jnp.zeros((CHUNK, D), jnp.float32)
    w1_pad = jnp.pad(W1, ((0, 0), (0, D - C_OUT)))
    b1_pad = jnp.pad(b1.reshape(1, C_OUT), ((0, 0), (0, D - C_OUT)))

    deg_s = _degrees_kernel(s_slab, zeros_row, ones_row)
    deg_r = _degrees_kernel(r_slab, zeros_row, ones_row)
    y, sr = _tc_encode(x_pad, W0, b0.reshape(1, D), deg_s, deg_r)
    acc1 = _agg128(y, s_flat, r_flat, zeros_row)
    z = _tc_decode(acc1, sr, w1_pad, b1_pad)
    acc2 = _agg128(z, s_flat, r_flat, zeros_row)
    out = _tc_combine(acc2)
    return out[:N_NODES]


# speedup vs baseline: 1.3431x; 1.0001x over previous
import functools

import jax
import jax.numpy as jnp
from jax import lax
from jax.experimental import pallas as pl
from jax.experimental.pallas import tpu as pltpu
from jax.experimental.pallas import tpu_sc as plsc

N_NODES = 10000
N_EDGES = 320000
D = 128
C_OUT = 16

NPAD = 10240
N_CORES = 2
N_SUB = 16
CHUNK = 128
CHUNKS_PER_TILE = 80
EPAD = N_CORES * N_SUB * CHUNKS_PER_TILE * CHUNK
ROWS_PER_TILE = NPAD // N_SUB
HOPS = ROWS_PER_TILE // CHUNK
PAD_IDX = N_NODES

_mesh = plsc.VectorSubcoreMesh(core_axis_name="c", subcore_axis_name="s")


@functools.partial(
    pl.kernel,
    mesh=_mesh,
    out_type=jax.ShapeDtypeStruct((N_CORES, NPAD, D), jnp.float32),
    scratch_types=[
        pltpu.VMEM((CHUNKS_PER_TILE, CHUNK), jnp.int32),
        pltpu.VMEM((CHUNK, D), jnp.float32),
        pltpu.VMEM((CHUNK, D), jnp.float32),
        pltpu.VMEM_SHARED((NPAD, D), jnp.float32),
    ],
)
def _degrees_kernel(i_hbm, zeros_hbm, ones_hbm, out_hbm,
                    idx, ones_v, stage, acc):
    c = lax.axis_index("c")
    s = lax.axis_index("s")
    pltpu.sync_copy(i_hbm.at[c, s], idx)
    pltpu.sync_copy(ones_hbm, ones_v)
    pltpu.sync_copy(zeros_hbm, stage)
    base = s * ROWS_PER_TILE
    for h in range(HOPS):
        pltpu.sync_copy(stage, acc.at[pl.ds(base + h * CHUNK, CHUNK)])
    plsc.subcore_barrier()

    @pl.loop(0, CHUNKS_PER_TILE)
    def _(j):
        pltpu.sync_copy(ones_v, acc.at[idx.at[j]], add=True)

    plsc.subcore_barrier()
    for h in range(HOPS):
        off = base + h * CHUNK
        pltpu.sync_copy(acc.at[pl.ds(off, CHUNK)], stage)
        pltpu.sync_copy(stage, out_hbm.at[c, pl.ds(off, CHUNK)])


TOTAL_CHUNKS = EPAD // CHUNK
KA = 152
KB = 8
KP = 8
A_TOTAL = N_SUB * KA
assert N_SUB * (KA + KB) == TOTAL_CHUNKS


@functools.partial(
    pl.kernel,
    mesh=_mesh,
    out_type=jax.ShapeDtypeStruct((N_CORES, NPAD, D), jnp.float32),
    scratch_types=[
        pltpu.VMEM((KP, CHUNK), jnp.int32),
        pltpu.VMEM((KP, CHUNK), jnp.int32),
        pltpu.VMEM((2, CHUNK, D), jnp.float32),
        pltpu.VMEM_SHARED((NPAD, D), jnp.float32),
        pltpu.SemaphoreType.DMA,
        pltpu.SemaphoreType.DMA,
    ],
)
def _agg128(table_hbm, s_hbm, r_hbm, zeros_hbm, out_hbm,
            sidx, ridx, rows, acc, gsem0, gsem1):
    c = lax.axis_index("c")
    s = lax.axis_index("s")
    base = s * ROWS_PER_TILE
    pltpu.sync_copy(zeros_hbm, rows.at[0])
    for h in range(HOPS):
        pltpu.sync_copy(rows.at[0], acc.at[pl.ds(base + h * CHUNK, CHUNK)])
    plsc.subcore_barrier()

    start = jnp.where(c == 0, s * KA, A_TOTAL + s * KB)
    nphases = jnp.where(c == 0, KA // KP, KB // KP)

    for p in range(KA // KP):
        @pl.when(p < nphases)
        def _():
            off = start + p * KP
            pltpu.sync_copy(s_hbm.at[pl.ds(off, KP)], sidx)
            pltpu.sync_copy(r_hbm.at[pl.ds(off, KP)], ridx)

            pltpu.async_copy(table_hbm.at[sidx.at[0]], rows.at[0], gsem0)

            @pl.loop(0, KP, step=2)
            def _(j):
                pltpu.async_copy(table_hbm.at[sidx.at[j + 1]], rows.at[1],
                                 gsem1)
                pltpu.make_async_copy(table_hbm.at[sidx.at[j]], rows.at[0],
                                      gsem0).wait()
                pltpu.sync_copy(rows.at[0], acc.at[ridx.at[j]], add=True)

                @pl.when(j + 2 < KP)
                def _():
                    pltpu.async_copy(table_hbm.at[sidx.at[j + 2]], rows.at[0],
                                     gsem0)

                pltpu.make_async_copy(table_hbm.at[sidx.at[j + 1]],
                                      rows.at[1], gsem1).wait()
                pltpu.sync_copy(rows.at[1], acc.at[ridx.at[j + 1]], add=True)

    plsc.subcore_barrier()
    for h in range(HOPS):
        off = base + h * CHUNK
        pltpu.sync_copy(acc.at[pl.ds(off, CHUNK)], rows.at[0])
        pltpu.sync_copy(rows.at[0], out_hbm.at[c, pl.ds(off, CHUNK)])


_BLK = 1024


def _tc_encode_body(x_ref, w0_ref, b0_ref, degs_ref, degr_ref, y_ref, sr_ref):
    ds = (degs_ref[0] + degs_ref[1])[:, 0:1]
    dr = (degr_ref[0] + degr_ref[1])[:, 0:1]
    ss = lax.rsqrt(jnp.maximum(ds, 1.0))
    sr = lax.rsqrt(jnp.maximum(dr, 1.0))
    y = jnp.dot(x_ref[...], w0_ref[...], preferred_element_type=jnp.float32)
    y = (y + b0_ref[...]) * ss
    row = pl.program_id(0) * _BLK + lax.broadcasted_iota(jnp.int32, (_BLK, 1), 0)
    y_ref[...] = jnp.where(row < N_NODES, y, 0.0)
    sr_ref[...] = sr


def _tc_encode(x_pad, w0, b0, deg_s, deg_r):
    return pl.pallas_call(
        _tc_encode_body,
        grid=(NPAD // _BLK,),
        in_specs=[
            pl.BlockSpec((_BLK, D), lambda i: (i, 0)),
            pl.BlockSpec((D, D), lambda i: (0, 0)),
            pl.BlockSpec((1, D), lambda i: (0, 0)),
            pl.BlockSpec((N_CORES, _BLK, D), lambda i: (0, i, 0)),
            pl.BlockSpec((N_CORES, _BLK, D), lambda i: (0, i, 0)),
        ],
        out_specs=[
            pl.BlockSpec((_BLK, D), lambda i: (i, 0)),
            pl.BlockSpec((_BLK, 1), lambda i: (i, 0)),
        ],
        out_shape=[
            jax.ShapeDtypeStruct((NPAD, D), jnp.float32),
            jax.ShapeDtypeStruct((NPAD, 1), jnp.float32),
        ],
    )(x_pad, w0, b0, deg_s, deg_r)


def _tc_decode_body(acc_ref, sr_ref, w1_ref, b1_ref, z_ref):
    h = (acc_ref[0] + acc_ref[1]) * sr_ref[...]
    h = jnp.where(h >= 0, h, 0.01 * h)
    z = jnp.dot(h, w1_ref[...], preferred_element_type=jnp.float32) + b1_ref[...]
    z = 1.0 / (1.0 + jnp.exp(-z))
    row = pl.program_id(0) * _BLK + lax.broadcasted_iota(jnp.int32, (_BLK, 1), 0)
    col = lax.broadcasted_iota(jnp.int32, (_BLK, D), 1)
    z_ref[...] = jnp.where((row < N_NODES) & (col < C_OUT), z, 0.0)


def _tc_decode(acc, sr, w1, b1):
    return pl.pallas_call(
        _tc_decode_body,
        grid=(NPAD // _BLK,),
        in_specs=[
            pl.BlockSpec((N_CORES, _BLK, D), lambda i: (0, i, 0)),
            pl.BlockSpec((_BLK, 1), lambda i: (i, 0)),
            pl.BlockSpec((D, D), lambda i: (0, 0)),
            pl.BlockSpec((1, D), lambda i: (0, 0)),
        ],
        out_specs=pl.BlockSpec((_BLK, D), lambda i: (i, 0)),
        out_shape=jax.ShapeDtypeStruct((NPAD, D), jnp.float32),
    )(acc, sr, w1, b1)


def _tc_combine_body(acc_ref, out_ref):
    out_ref[...] = (acc_ref[0] + acc_ref[1])[:, :C_OUT]


def _tc_combine(acc):
    return pl.pallas_call(
        _tc_combine_body,
        grid=(NPAD // _BLK,),
        in_specs=[pl.BlockSpec((N_CORES, _BLK, D), lambda i: (0, i, 0))],
        out_specs=pl.BlockSpec((_BLK, C_OUT), lambda i: (i, 0)),
        out_shape=jax.ShapeDtypeStruct((NPAD, C_OUT), jnp.float32),
    )(acc)


def kernel(x, edge_index, W0, b0, W1, b1):
    senders = edge_index[0].astype(jnp.int32)
    receivers = edge_index[1].astype(jnp.int32)
    pad = EPAD - N_EDGES
    senders = jnp.pad(senders, (0, pad), constant_values=PAD_IDX)
    receivers = jnp.pad(receivers, (0, pad), constant_values=PAD_IDX)
    s_slab = senders.reshape(N_CORES, N_SUB, CHUNKS_PER_TILE, CHUNK)
    r_slab = receivers.reshape(N_CORES, N_SUB, CHUNKS_PER_TILE, CHUNK)
    s_flat = senders.reshape(TOTAL_CHUNKS, CHUNK)
    r_flat = receivers.reshape(TOTAL_CHUNKS, CHUNK)

    x_pad = jnp.pad(x, ((0, NPAD - N_NODES), (0, 0)))
    ones_row = jnp.ones((CHUNK, D), jnp.float32)
    zeros_row = jnp.zeros((CHUNK, D), jnp.float32)
    w1_pad = jnp.pad(W1, ((0, 0), (0, D - C_OUT)))
    b1_pad = jnp.pad(b1.reshape(1, C_OUT), ((0, 0), (0, D - C_OUT)))

    deg_s = _degrees_kernel(s_slab, zeros_row, ones_row)
    deg_r = _degrees_kernel(r_slab, zeros_row, ones_row)
    y, sr = _tc_encode(x_pad, W0, b0.reshape(1, D), deg_s, deg_r)
    acc1 = _agg128(y, s_flat, r_flat, zeros_row)
    z = _tc_decode(acc1, sr, w1_pad, b1_pad)
    acc2 = _agg128(z, s_flat, r_flat, zeros_row)
    out = _tc_combine(acc2)
    return out[:N_NODES]
